# baseline (speedup 1.0000x reference)
import functools

import jax
import jax.numpy as jnp
from jax import lax
from jax.experimental import pallas as pl
from jax.experimental.pallas import tpu as pltpu
from jax.experimental.pallas import tpu_sc as plsc

D = 128
LANES = 16
NSLICE = D // LANES
NC, NS = 2, 16
NW = NC * NS

N_ATOMS = 10000
N_BONDS = 320000
MAX_NB = 32

GS_A = 320
N_ATOMS_PAD = NW * GS_A
GS_G = 1
GS_R = GS_G * MAX_NB
GS_NG = GS_A // GS_G

BM_B = N_BONDS // NW
BM_C = 40
BM_NCH = BM_B // BM_C


def _mm_relu_body(x_ref, w_ref, o_ref):
    o_ref[...] = jnp.maximum(
        jnp.dot(x_ref[...], w_ref[...], preferred_element_type=jnp.float32), 0.0)


def _mm_relu(x, w, blk):
    n = x.shape[0]
    return pl.pallas_call(
        _mm_relu_body,
        grid=(n // blk,),
        in_specs=[pl.BlockSpec((blk, D), lambda i: (i, 0)),
                  pl.BlockSpec((D, D), lambda i: (0, 0))],
        out_specs=pl.BlockSpec((blk, D), lambda i: (i, 0)),
        out_shape=jax.ShapeDtypeStruct((n, D), jnp.float32),
    )(x, w)


def _mm2_body(x_ref, wi_ref, wh_ref, o_ref):
    t = jnp.maximum(
        jnp.dot(x_ref[...], wi_ref[...], preferred_element_type=jnp.float32), 0.0)
    o_ref[...] = jnp.dot(t, wh_ref[...], preferred_element_type=jnp.float32)


def _mm2(x, wi, wh, blk):
    n = x.shape[0]
    return pl.pallas_call(
        _mm2_body,
        grid=(n // blk,),
        in_specs=[pl.BlockSpec((blk, D), lambda i: (i, 0)),
                  pl.BlockSpec((D, D), lambda i: (0, 0)),
                  pl.BlockSpec((D, D), lambda i: (0, 0))],
        out_specs=pl.BlockSpec((blk, D), lambda i: (i, 0)),
        out_shape=jax.ShapeDtypeStruct((n, D), jnp.float32),
    )(x, wi, wh)


def _add_mm_relu_body(x_ref, y_ref, w_ref, o_ref):
    o_ref[...] = jnp.maximum(
        jnp.dot(x_ref[...] + y_ref[...], w_ref[...],
                preferred_element_type=jnp.float32), 0.0)


def _add_mm_relu(x, y, w, blk):
    n = x.shape[0]
    return pl.pallas_call(
        _add_mm_relu_body,
        grid=(n // blk,),
        in_specs=[pl.BlockSpec((blk, D), lambda i: (i, 0)),
                  pl.BlockSpec((blk, D), lambda i: (i, 0)),
                  pl.BlockSpec((D, D), lambda i: (0, 0))],
        out_specs=pl.BlockSpec((blk, D), lambda i: (i, 0)),
        out_shape=jax.ShapeDtypeStruct((n, D), jnp.float32),
    )(x, y, w)


GS_NBUF = 4
GS_OCH = GS_NBUF * GS_G


def _gs_reduce_atom(buf, r0, ov, orow):
    sls = [pl.ds(j * LANES, LANES) for j in range(NSLICE)]
    accs = tuple(buf[r0, sl] + buf[r0 + 1, sl] + buf[r0 + 2, sl]
                 + buf[r0 + 3, sl] for sl in sls)

    def red(t, accs):
        b4 = r0 + t * 4
        return tuple(accs[j] + buf[b4, sls[j]] + buf[b4 + 1, sls[j]]
                     + buf[b4 + 2, sls[j]] + buf[b4 + 3, sls[j]]
                     for j in range(NSLICE))

    accs = lax.fori_loop(1, MAX_NB // 4, red, accs)
    for j in range(NSLICE):
        ov[orow, sls[j]] = accs[j]


def _gs_body(table, idxh, out, idx_v, buf0, buf1, buf2, buf3, ov0, ov1,
             sg0, sg1, sg2, sg3, so0, so1):
    c = lax.axis_index("c")
    s = lax.axis_index("s")
    w = s * NC + c
    base_atom = w * GS_A
    pltpu.sync_copy(idxh.at[pl.ds(base_atom * MAX_NB, GS_A * MAX_NB)], idx_v)

    bufs = (buf0, buf1, buf2, buf3)
    sgs = (sg0, sg1, sg2, sg3)
    ovs = (ov0, ov1)
    sos = (so0, so1)

    def issue_gather(g, b):
        pltpu.async_copy(table.at[idx_v.at[pl.ds(g * GS_R, GS_R)]],
                         bufs[b], sgs[b])

    for b in range(GS_NBUF):
        issue_gather(b, b)

    def outer(g3, _):
        for half in range(2):
            ov = ovs[half]
            orow0 = base_atom + g3 * 2 * GS_OCH + half * GS_OCH

            @pl.when(g3 > 0)
            def _():
                pltpu.make_async_copy(
                    ov, out.at[pl.ds(orow0 - 2 * GS_OCH, GS_OCH)],
                    sos[half]).wait()

            for b in range(GS_NBUF):
                g = g3 * 2 * GS_NBUF + half * GS_NBUF + b
                pltpu.make_async_copy(table.at[idx_v.at[pl.ds(0, GS_R)]],
                                      bufs[b], sgs[b]).wait()
                for a in range(GS_G):
                    _gs_reduce_atom(bufs[b], a * MAX_NB, ov, b * GS_G + a)

                @pl.when(g + GS_NBUF < GS_NG)
                def _():
                    issue_gather(g + GS_NBUF, b)

            pltpu.async_copy(ov, out.at[pl.ds(orow0, GS_OCH)], sos[half])
        return 0

    lax.fori_loop(0, GS_NG // (2 * GS_NBUF), outer, 0)
    for half in range(2):
        pltpu.make_async_copy(
            ovs[half],
            out.at[pl.ds(base_atom + GS_A - (2 - half) * GS_OCH, GS_OCH)],
            sos[half]).wait()


def _gather_sum(table, idx_flat):
    mesh = plsc.VectorSubcoreMesh(core_axis_name="c", subcore_axis_name="s")
    return pl.kernel(
        _gs_body,
        out_type=jax.ShapeDtypeStruct((N_ATOMS_PAD, D), jnp.float32),
        mesh=mesh,
        scratch_types=[
            pltpu.VMEM((GS_A * MAX_NB,), jnp.int32),
            pltpu.VMEM((GS_R, D), jnp.float32),
            pltpu.VMEM((GS_R, D), jnp.float32),
            pltpu.VMEM((GS_R, D), jnp.float32),
            pltpu.VMEM((GS_R, D), jnp.float32),
            pltpu.VMEM((GS_OCH, D), jnp.float32),
            pltpu.VMEM((GS_OCH, D), jnp.float32),
            pltpu.SemaphoreType.DMA,
            pltpu.SemaphoreType.DMA,
            pltpu.SemaphoreType.DMA,
            pltpu.SemaphoreType.DMA,
            pltpu.SemaphoreType.DMA,
            pltpu.SemaphoreType.DMA,
        ],
    )(table, idx_flat)


DG_G = 4
DG_R = DG_G * MAX_NB
DG_OCH = 8
DG_NG = GS_A // DG_G


def _dg_body(ta, tb, idxa_h, idxb_h, outa, outb,
             idxa_v, idxb_v, a0, a1, b0, b1, ova0, ova1, ovb0, ovb1,
             sa0, sa1, sb0, sb1, soa0, soa1, sob0, sob1):
    c = lax.axis_index("c")
    s = lax.axis_index("s")
    w = s * NC + c
    base_atom = w * GS_A
    pltpu.sync_copy(idxa_h.at[pl.ds(base_atom * MAX_NB, GS_A * MAX_NB)],
                    idxa_v)
    pltpu.sync_copy(idxb_h.at[pl.ds(base_atom * MAX_NB, GS_A * MAX_NB)],
                    idxb_v)

    bufa = (a0, a1)
    bufb = (b0, b1)
    sga = (sa0, sa1)
    sgb = (sb0, sb1)
    ovas = (ova0, ova1)
    ovbs = (ovb0, ovb1)
    soas = (soa0, soa1)
    sobs = (sob0, sob1)

    def issue(g, b):
        pltpu.async_copy(ta.at[idxa_v.at[pl.ds(g * DG_R, DG_R)]],
                         bufa[b], sga[b])
        pltpu.async_copy(tb.at[idxb_v.at[pl.ds(g * DG_R, DG_R)]],
                         bufb[b], sgb[b])

    issue(0, 0)
    issue(1, 1)

    def outer(g3, _):
        for half in range(2):
            ova, ovb = ovas[half], ovbs[half]
            orow0 = base_atom + g3 * 2 * DG_OCH + half * DG_OCH

            @pl.when(g3 > 0)
            def _():
                pltpu.make_async_copy(
                    ova, outa.at[pl.ds(orow0 - 2 * DG_OCH, DG_OCH)],
                    soas[half]).wait()
                pltpu.make_async_copy(
                    ovb, outb.at[pl.ds(orow0 - 2 * DG_OCH, DG_OCH)],
                    sobs[half]).wait()

            for q in range(2):
                g = g3 * 4 + half * 2 + q
                pltpu.make_async_copy(ta.at[idxa_v.at[pl.ds(0, DG_R)]],
                                      bufa[q], sga[q]).wait()
                pltpu.make_async_copy(tb.at[idxb_v.at[pl.ds(0, DG_R)]],
                                      bufb[q], sgb[q]).wait()
                for a in range(DG_G):
                    orow = q * DG_G + a
                    _gs_reduce_atom(bufa[q], a * MAX_NB, ova, orow)
                    _gs_reduce_atom(bufb[q], a * MAX_NB, ovb, orow)

                @pl.when(g + 2 < DG_NG)
                def _():
                    issue(g + 2, q)

            pltpu.async_copy(ova, outa.at[pl.ds(orow0, DG_OCH)], soas[half])
            pltpu.async_copy(ovb, outb.at[pl.ds(orow0, DG_OCH)], sobs[half])
        return 0

    lax.fori_loop(0, DG_NG // 4, outer, 0)
    for half in range(2):
        row = base_atom + GS_A - (2 - half) * DG_OCH
        pltpu.make_async_copy(ovas[half], outa.at[pl.ds(row, DG_OCH)],
                              soas[half]).wait()
        pltpu.make_async_copy(ovbs[half], outb.at[pl.ds(row, DG_OCH)],
                              sobs[half]).wait()


def _dual_gather_sum(ta, tb, idxa_flat, idxb_flat):
    mesh = plsc.VectorSubcoreMesh(core_axis_name="c", subcore_axis_name="s")
    return pl.kernel(
        _dg_body,
        out_type=(jax.ShapeDtypeStruct((N_ATOMS_PAD, D), jnp.float32),
                  jax.ShapeDtypeStruct((N_ATOMS_PAD, D), jnp.float32)),
        mesh=mesh,
        scratch_types=[
            pltpu.VMEM((GS_A * MAX_NB,), jnp.int32),
            pltpu.VMEM((GS_A * MAX_NB,), jnp.int32),
            pltpu.VMEM((DG_R, D), jnp.float32),
            pltpu.VMEM((DG_R, D), jnp.float32),
            pltpu.VMEM((DG_R, D), jnp.float32),
            pltpu.VMEM((DG_R, D), jnp.float32),
            pltpu.VMEM((DG_OCH, D), jnp.float32),
            pltpu.VMEM((DG_OCH, D), jnp.float32),
            pltpu.VMEM((DG_OCH, D), jnp.float32),
            pltpu.VMEM((DG_OCH, D), jnp.float32),
            pltpu.SemaphoreType.DMA,
            pltpu.SemaphoreType.DMA,
            pltpu.SemaphoreType.DMA,
            pltpu.SemaphoreType.DMA,
            pltpu.SemaphoreType.DMA,
            pltpu.SemaphoreType.DMA,
            pltpu.SemaphoreType.DMA,
            pltpu.SemaphoreType.DMA,
        ],
    )(ta, tb, idxa_flat, idxb_flat)


def _bm_body(amsg, bonds, idxa_h, idxb_h, out,
             idxa_v, idxb_v, bufa0, bufb0, bufa1, bufb1, out_v,
             sa0, sb0, sa1, sb1):
    w = lax.axis_index("s") * NC + lax.axis_index("c")
    base = w * BM_B
    pltpu.sync_copy(idxa_h.at[pl.ds(base, BM_B)], idxa_v)
    pltpu.sync_copy(idxb_h.at[pl.ds(base, BM_B)], idxb_v)
    bufsa = (bufa0, bufa1)
    bufsb = (bufb0, bufb1)
    semsa = (sa0, sa1)
    semsb = (sb0, sb1)

    def issue(g, b):
        pltpu.async_copy(amsg.at[idxa_v.at[pl.ds(g * BM_C, BM_C)]],
                         bufsa[b], semsa[b])
        pltpu.async_copy(bonds.at[idxb_v.at[pl.ds(g * BM_C, BM_C)]],
                         bufsb[b], semsb[b])

    issue(0, 0)
    issue(1, 1)

    def outer(g2, _):
        for b in range(2):
            g = g2 * 2 + b
            ba, bb = bufsa[b], bufsb[b]
            pltpu.make_async_copy(amsg.at[idxa_v.at[pl.ds(0, BM_C)]],
                                  ba, semsa[b]).wait()
            pltpu.make_async_copy(bonds.at[idxb_v.at[pl.ds(0, BM_C)]],
                                  bb, semsb[b]).wait()

            def row_body(r, _):
                for j in range(NSLICE):
                    s = pl.ds(j * LANES, LANES)
                    out_v[r, s] = jnp.maximum(ba[r, s] - bb[r, s], 0.0)
                return 0

            lax.fori_loop(0, BM_C, row_body, 0)
            pltpu.sync_copy(out_v, out.at[pl.ds(base + g * BM_C, BM_C)])

            @pl.when(g + 2 < BM_NCH)
            def _():
                issue(g + 2, b)
        return 0

    lax.fori_loop(0, BM_NCH // 2, outer, 0)


def _bond_msg(amsg, bonds, b2a, b2revb):
    mesh = plsc.VectorSubcoreMesh(core_axis_name="c", subcore_axis_name="s")
    return pl.kernel(
        _bm_body,
        out_type=jax.ShapeDtypeStruct((N_BONDS, D), jnp.float32),
        mesh=mesh,
        scratch_types=[
            pltpu.VMEM((BM_B,), jnp.int32),
            pltpu.VMEM((BM_B,), jnp.int32),
            pltpu.VMEM((BM_C, D), jnp.float32),
            pltpu.VMEM((BM_C, D), jnp.float32),
            pltpu.VMEM((BM_C, D), jnp.float32),
            pltpu.VMEM((BM_C, D), jnp.float32),
            pltpu.VMEM((BM_C, D), jnp.float32),
            pltpu.SemaphoreType.DMA,
            pltpu.SemaphoreType.DMA,
            pltpu.SemaphoreType.DMA,
            pltpu.SemaphoreType.DMA,
        ],
    )(amsg, bonds, b2a, b2revb)


def _pad_idx(idx2d):
    flat = idx2d.reshape(-1).astype(jnp.int32)
    pad = N_ATOMS_PAD * MAX_NB - flat.shape[0]
    return jnp.pad(flat, (0, pad))


def kernel(f_atoms, f_bonds, a2b, b2a, b2revb, a_scope, b_scope, a2a,
           features_batch, W_i_atom, W_h_atom, W_i_bond, W_h_bond):
    input_atom = _mm_relu(f_atoms, W_i_atom, blk=1000)
    BW = _mm2(f_bonds, W_i_bond, W_h_bond, blk=3200)
    nei_pad, AW = _dual_gather_sum(input_atom, BW,
                                   _pad_idx(a2a), _pad_idx(a2b))
    nei_a_msg = nei_pad[:N_ATOMS]
    bond_output = _bond_msg(AW, BW,
                            b2a.astype(jnp.int32), b2revb.astype(jnp.int32))

    node_output = _add_mm_relu(input_atom, nei_a_msg, W_h_atom, blk=1000)
    return (node_output, bond_output)


# speedup vs baseline: 1.0612x; 1.0612x over previous
import functools

import jax
import jax.numpy as jnp
from jax import lax
from jax.experimental import pallas as pl
from jax.experimental.pallas import tpu as pltpu
from jax.experimental.pallas import tpu_sc as plsc

D = 128
LANES = 16
NSLICE = D // LANES
NC, NS = 2, 16
NW = NC * NS

N_ATOMS = 10000
N_BONDS = 320000
MAX_NB = 32

GS_A = 320
N_ATOMS_PAD = NW * GS_A
GS_G = 1
GS_R = GS_G * MAX_NB
GS_NG = GS_A // GS_G

BM_B = N_BONDS // NW
BM_C = 40
BM_NCH = BM_B // BM_C


def _mm_relu_body(x_ref, w_ref, o_ref):
    o_ref[...] = jnp.maximum(
        jnp.dot(x_ref[...], w_ref[...], preferred_element_type=jnp.float32), 0.0)


def _mm_relu(x, w, blk):
    n = x.shape[0]
    return pl.pallas_call(
        _mm_relu_body,
        grid=(n // blk,),
        in_specs=[pl.BlockSpec((blk, D), lambda i: (i, 0)),
                  pl.BlockSpec((D, D), lambda i: (0, 0))],
        out_specs=pl.BlockSpec((blk, D), lambda i: (i, 0)),
        out_shape=jax.ShapeDtypeStruct((n, D), jnp.float32),
    )(x, w)


def _mm2_body(x_ref, wi_ref, wh_ref, o_ref):
    t = jnp.maximum(
        jnp.dot(x_ref[...], wi_ref[...], preferred_element_type=jnp.float32), 0.0)
    o_ref[...] = jnp.dot(t, wh_ref[...], preferred_element_type=jnp.float32)


def _mm2(x, wi, wh, blk):
    n = x.shape[0]
    return pl.pallas_call(
        _mm2_body,
        grid=(n // blk,),
        in_specs=[pl.BlockSpec((blk, D), lambda i: (i, 0)),
                  pl.BlockSpec((D, D), lambda i: (0, 0)),
                  pl.BlockSpec((D, D), lambda i: (0, 0))],
        out_specs=pl.BlockSpec((blk, D), lambda i: (i, 0)),
        out_shape=jax.ShapeDtypeStruct((n, D), jnp.float32),
    )(x, wi, wh)


def _add_mm_relu_body(x_ref, y_ref, w_ref, o_ref):
    o_ref[...] = jnp.maximum(
        jnp.dot(x_ref[...] + y_ref[...], w_ref[...],
                preferred_element_type=jnp.float32), 0.0)


def _add_mm_relu(x, y, w, blk):
    n = x.shape[0]
    return pl.pallas_call(
        _add_mm_relu_body,
        grid=(n // blk,),
        in_specs=[pl.BlockSpec((blk, D), lambda i: (i, 0)),
                  pl.BlockSpec((blk, D), lambda i: (i, 0)),
                  pl.BlockSpec((D, D), lambda i: (0, 0))],
        out_specs=pl.BlockSpec((blk, D), lambda i: (i, 0)),
        out_shape=jax.ShapeDtypeStruct((n, D), jnp.float32),
    )(x, y, w)


GS_NBUF = 4
GS_OCH = GS_NBUF * GS_G


def _gs_reduce_atom(buf, r0, ov, orow):
    sls = [pl.ds(j * LANES, LANES) for j in range(NSLICE)]
    accs = tuple(buf[r0, sl] + buf[r0 + 1, sl] + buf[r0 + 2, sl]
                 + buf[r0 + 3, sl] for sl in sls)

    def red(t, accs):
        b4 = r0 + t * 4
        return tuple(accs[j] + buf[b4, sls[j]] + buf[b4 + 1, sls[j]]
                     + buf[b4 + 2, sls[j]] + buf[b4 + 3, sls[j]]
                     for j in range(NSLICE))

    accs = lax.fori_loop(1, MAX_NB // 4, red, accs)
    for j in range(NSLICE):
        ov[orow, sls[j]] = accs[j]


def _gs_body(table, idxh, out, idx_v, buf0, buf1, buf2, buf3, ov0, ov1,
             sg0, sg1, sg2, sg3, so0, so1):
    c = lax.axis_index("c")
    s = lax.axis_index("s")
    w = s * NC + c
    base_atom = w * GS_A
    pltpu.sync_copy(idxh.at[pl.ds(base_atom * MAX_NB, GS_A * MAX_NB)], idx_v)

    bufs = (buf0, buf1, buf2, buf3)
    sgs = (sg0, sg1, sg2, sg3)
    ovs = (ov0, ov1)
    sos = (so0, so1)

    def issue_gather(g, b):
        pltpu.async_copy(table.at[idx_v.at[pl.ds(g * GS_R, GS_R)]],
                         bufs[b], sgs[b])

    for b in range(GS_NBUF):
        issue_gather(b, b)

    def outer(g3, _):
        for half in range(2):
            ov = ovs[half]
            orow0 = base_atom + g3 * 2 * GS_OCH + half * GS_OCH

            @pl.when(g3 > 0)
            def _():
                pltpu.make_async_copy(
                    ov, out.at[pl.ds(orow0 - 2 * GS_OCH, GS_OCH)],
                    sos[half]).wait()

            for b in range(GS_NBUF):
                g = g3 * 2 * GS_NBUF + half * GS_NBUF + b
                pltpu.make_async_copy(table.at[idx_v.at[pl.ds(0, GS_R)]],
                                      bufs[b], sgs[b]).wait()
                for a in range(GS_G):
                    _gs_reduce_atom(bufs[b], a * MAX_NB, ov, b * GS_G + a)

                @pl.when(g + GS_NBUF < GS_NG)
                def _():
                    issue_gather(g + GS_NBUF, b)

            pltpu.async_copy(ov, out.at[pl.ds(orow0, GS_OCH)], sos[half])
        return 0

    lax.fori_loop(0, GS_NG // (2 * GS_NBUF), outer, 0)
    for half in range(2):
        pltpu.make_async_copy(
            ovs[half],
            out.at[pl.ds(base_atom + GS_A - (2 - half) * GS_OCH, GS_OCH)],
            sos[half]).wait()


def _gather_sum(table, idx_flat):
    mesh = plsc.VectorSubcoreMesh(core_axis_name="c", subcore_axis_name="s")
    return pl.kernel(
        _gs_body,
        out_type=jax.ShapeDtypeStruct((N_ATOMS_PAD, D), jnp.float32),
        mesh=mesh,
        scratch_types=[
            pltpu.VMEM((GS_A * MAX_NB,), jnp.int32),
            pltpu.VMEM((GS_R, D), jnp.float32),
            pltpu.VMEM((GS_R, D), jnp.float32),
            pltpu.VMEM((GS_R, D), jnp.float32),
            pltpu.VMEM((GS_R, D), jnp.float32),
            pltpu.VMEM((GS_OCH, D), jnp.float32),
            pltpu.VMEM((GS_OCH, D), jnp.float32),
            pltpu.SemaphoreType.DMA,
            pltpu.SemaphoreType.DMA,
            pltpu.SemaphoreType.DMA,
            pltpu.SemaphoreType.DMA,
            pltpu.SemaphoreType.DMA,
            pltpu.SemaphoreType.DMA,
        ],
    )(table, idx_flat)


DG_G = 4
DG_R = DG_G * MAX_NB
DG_OCH = 8
DG_PA0 = 480
DG_PA1 = 160
DG_IDX_ROWS = NS * DG_PA0 + (NS - 1) * DG_PA1 + DG_PA0


def _dg_body(ta, tb, idxa_h, idxb_h, outa, outb,
             idxa_v, idxb_v, a0, a1, b0, b1, ova0, ova1, ovb0, ovb1,
             sa0, sa1, sb0, sb1, soa0, soa1, sob0, sob1):
    c = lax.axis_index("c")
    s = lax.axis_index("s")
    pa = jnp.where(c == 0, DG_PA0, DG_PA1)
    ng = pa // DG_G
    base_atom = jnp.where(c == 0, s * DG_PA0, NS * DG_PA0 + s * DG_PA1)
    pltpu.sync_copy(idxa_h.at[pl.ds(base_atom * MAX_NB, DG_PA0 * MAX_NB)],
                    idxa_v)
    pltpu.sync_copy(idxb_h.at[pl.ds(base_atom * MAX_NB, DG_PA0 * MAX_NB)],
                    idxb_v)

    bufa = (a0, a1)
    bufb = (b0, b1)
    sga = (sa0, sa1)
    sgb = (sb0, sb1)
    ovas = (ova0, ova1)
    ovbs = (ovb0, ovb1)
    soas = (soa0, soa1)
    sobs = (sob0, sob1)

    def issue(g, b):
        pltpu.async_copy(ta.at[idxa_v.at[pl.ds(g * DG_R, DG_R)]],
                         bufa[b], sga[b])
        pltpu.async_copy(tb.at[idxb_v.at[pl.ds(g * DG_R, DG_R)]],
                         bufb[b], sgb[b])

    issue(0, 0)
    issue(1, 1)

    def outer(g3, _):
        for half in range(2):
            ova, ovb = ovas[half], ovbs[half]
            orow0 = base_atom + g3 * 2 * DG_OCH + half * DG_OCH

            @pl.when(g3 > 0)
            def _():
                pltpu.make_async_copy(
                    ova, outa.at[pl.ds(orow0 - 2 * DG_OCH, DG_OCH)],
                    soas[half]).wait()
                pltpu.make_async_copy(
                    ovb, outb.at[pl.ds(orow0 - 2 * DG_OCH, DG_OCH)],
                    sobs[half]).wait()

            for q in range(2):
                g = g3 * 4 + half * 2 + q
                pltpu.make_async_copy(ta.at[idxa_v.at[pl.ds(0, DG_R)]],
                                      bufa[q], sga[q]).wait()
                pltpu.make_async_copy(tb.at[idxb_v.at[pl.ds(0, DG_R)]],
                                      bufb[q], sgb[q]).wait()
                for a in range(DG_G):
                    orow = q * DG_G + a
                    _gs_reduce_atom(bufa[q], a * MAX_NB, ova, orow)
                    _gs_reduce_atom(bufb[q], a * MAX_NB, ovb, orow)

                @pl.when(g + 2 < ng)
                def _():
                    issue(g + 2, q)

            pltpu.async_copy(ova, outa.at[pl.ds(orow0, DG_OCH)], soas[half])
            pltpu.async_copy(ovb, outb.at[pl.ds(orow0, DG_OCH)], sobs[half])
        return 0

    lax.fori_loop(0, ng // 4, outer, 0)
    for half in range(2):
        row = base_atom + pa - (2 - half) * DG_OCH
        pltpu.make_async_copy(ovas[half], outa.at[pl.ds(row, DG_OCH)],
                              soas[half]).wait()
        pltpu.make_async_copy(ovbs[half], outb.at[pl.ds(row, DG_OCH)],
                              sobs[half]).wait()


def _dual_gather_sum(ta, tb, idxa_flat, idxb_flat):
    mesh = plsc.VectorSubcoreMesh(core_axis_name="c", subcore_axis_name="s")
    return pl.kernel(
        _dg_body,
        out_type=(jax.ShapeDtypeStruct((N_ATOMS_PAD, D), jnp.float32),
                  jax.ShapeDtypeStruct((N_ATOMS_PAD, D), jnp.float32)),
        mesh=mesh,
        scratch_types=[
            pltpu.VMEM((DG_PA0 * MAX_NB,), jnp.int32),
            pltpu.VMEM((DG_PA0 * MAX_NB,), jnp.int32),
            pltpu.VMEM((DG_R, D), jnp.float32),
            pltpu.VMEM((DG_R, D), jnp.float32),
            pltpu.VMEM((DG_R, D), jnp.float32),
            pltpu.VMEM((DG_R, D), jnp.float32),
            pltpu.VMEM((DG_OCH, D), jnp.float32),
            pltpu.VMEM((DG_OCH, D), jnp.float32),
            pltpu.VMEM((DG_OCH, D), jnp.float32),
            pltpu.VMEM((DG_OCH, D), jnp.float32),
            pltpu.SemaphoreType.DMA,
            pltpu.SemaphoreType.DMA,
            pltpu.SemaphoreType.DMA,
            pltpu.SemaphoreType.DMA,
            pltpu.SemaphoreType.DMA,
            pltpu.SemaphoreType.DMA,
            pltpu.SemaphoreType.DMA,
            pltpu.SemaphoreType.DMA,
        ],
    )(ta, tb, idxa_flat, idxb_flat)


def _bm_body(amsg, bonds, idxa_h, idxb_h, out,
             idxa_v, idxb_v, bufa0, bufb0, bufa1, bufb1, out_v,
             sa0, sb0, sa1, sb1):
    w = lax.axis_index("s") * NC + lax.axis_index("c")
    base = w * BM_B
    pltpu.sync_copy(idxa_h.at[pl.ds(base, BM_B)], idxa_v)
    pltpu.sync_copy(idxb_h.at[pl.ds(base, BM_B)], idxb_v)
    bufsa = (bufa0, bufa1)
    bufsb = (bufb0, bufb1)
    semsa = (sa0, sa1)
    semsb = (sb0, sb1)

    def issue(g, b):
        pltpu.async_copy(amsg.at[idxa_v.at[pl.ds(g * BM_C, BM_C)]],
                         bufsa[b], semsa[b])
        pltpu.async_copy(bonds.at[idxb_v.at[pl.ds(g * BM_C, BM_C)]],
                         bufsb[b], semsb[b])

    issue(0, 0)
    issue(1, 1)

    def outer(g2, _):
        for b in range(2):
            g = g2 * 2 + b
            ba, bb = bufsa[b], bufsb[b]
            pltpu.make_async_copy(amsg.at[idxa_v.at[pl.ds(0, BM_C)]],
                                  ba, semsa[b]).wait()
            pltpu.make_async_copy(bonds.at[idxb_v.at[pl.ds(0, BM_C)]],
                                  bb, semsb[b]).wait()

            def row_body(r, _):
                for j in range(NSLICE):
                    s = pl.ds(j * LANES, LANES)
                    out_v[r, s] = jnp.maximum(ba[r, s] - bb[r, s], 0.0)
                return 0

            lax.fori_loop(0, BM_C, row_body, 0)
            pltpu.sync_copy(out_v, out.at[pl.ds(base + g * BM_C, BM_C)])

            @pl.when(g + 2 < BM_NCH)
            def _():
                issue(g + 2, b)
        return 0

    lax.fori_loop(0, BM_NCH // 2, outer, 0)


def _bond_msg(amsg, bonds, b2a, b2revb):
    mesh = plsc.VectorSubcoreMesh(core_axis_name="c", subcore_axis_name="s")
    return pl.kernel(
        _bm_body,
        out_type=jax.ShapeDtypeStruct((N_BONDS, D), jnp.float32),
        mesh=mesh,
        scratch_types=[
            pltpu.VMEM((BM_B,), jnp.int32),
            pltpu.VMEM((BM_B,), jnp.int32),
            pltpu.VMEM((BM_C, D), jnp.float32),
            pltpu.VMEM((BM_C, D), jnp.float32),
            pltpu.VMEM((BM_C, D), jnp.float32),
            pltpu.VMEM((BM_C, D), jnp.float32),
            pltpu.VMEM((BM_C, D), jnp.float32),
            pltpu.SemaphoreType.DMA,
            pltpu.SemaphoreType.DMA,
            pltpu.SemaphoreType.DMA,
            pltpu.SemaphoreType.DMA,
        ],
    )(amsg, bonds, b2a, b2revb)


def _pad_idx(idx2d):
    flat = idx2d.reshape(-1).astype(jnp.int32)
    pad = DG_IDX_ROWS * MAX_NB - flat.shape[0]
    return jnp.pad(flat, (0, pad))


def kernel(f_atoms, f_bonds, a2b, b2a, b2revb, a_scope, b_scope, a2a,
           features_batch, W_i_atom, W_h_atom, W_i_bond, W_h_bond):
    input_atom = _mm_relu(f_atoms, W_i_atom, blk=1000)
    BW = _mm2(f_bonds, W_i_bond, W_h_bond, blk=3200)
    nei_pad, AW = _dual_gather_sum(input_atom, BW,
                                   _pad_idx(a2a), _pad_idx(a2b))
    nei_a_msg = nei_pad[:N_ATOMS]
    bond_output = _bond_msg(AW, BW,
                            b2a.astype(jnp.int32), b2revb.astype(jnp.int32))

    node_output = _add_mm_relu(input_atom, nei_a_msg, W_h_atom, blk=1000)
    return (node_output, bond_output)


# speedup vs baseline: 1.0730x; 1.0112x over previous
import functools

import jax
import jax.numpy as jnp
from jax import lax
from jax.experimental import pallas as pl
from jax.experimental.pallas import tpu as pltpu
from jax.experimental.pallas import tpu_sc as plsc

D = 128
LANES = 16
NSLICE = D // LANES
NC, NS = 2, 16
NW = NC * NS

N_ATOMS = 10000
N_BONDS = 320000
MAX_NB = 32

GS_A = 320
N_ATOMS_PAD = NW * GS_A
GS_G = 1
GS_R = GS_G * MAX_NB
GS_NG = GS_A // GS_G

BM_B = N_BONDS // NW
BM_C = 40
BM_NCH = BM_B // BM_C


def _mm_relu_body(x_ref, w_ref, o_ref):
    o_ref[...] = jnp.maximum(
        jnp.dot(x_ref[...], w_ref[...], preferred_element_type=jnp.float32), 0.0)


def _mm_relu(x, w, blk):
    n = x.shape[0]
    return pl.pallas_call(
        _mm_relu_body,
        grid=(n // blk,),
        in_specs=[pl.BlockSpec((blk, D), lambda i: (i, 0)),
                  pl.BlockSpec((D, D), lambda i: (0, 0))],
        out_specs=pl.BlockSpec((blk, D), lambda i: (i, 0)),
        out_shape=jax.ShapeDtypeStruct((n, D), jnp.float32),
    )(x, w)


def _mm2_body(x_ref, wi_ref, wh_ref, o_ref):
    t = jnp.maximum(
        jnp.dot(x_ref[...], wi_ref[...], preferred_element_type=jnp.float32), 0.0)
    o_ref[...] = jnp.dot(t, wh_ref[...], preferred_element_type=jnp.float32)


def _mm2(x, wi, wh, blk):
    n = x.shape[0]
    return pl.pallas_call(
        _mm2_body,
        grid=(n // blk,),
        in_specs=[pl.BlockSpec((blk, D), lambda i: (i, 0)),
                  pl.BlockSpec((D, D), lambda i: (0, 0)),
                  pl.BlockSpec((D, D), lambda i: (0, 0))],
        out_specs=pl.BlockSpec((blk, D), lambda i: (i, 0)),
        out_shape=jax.ShapeDtypeStruct((n, D), jnp.float32),
    )(x, wi, wh)


def _add_mm_relu_body(x_ref, y_ref, w_ref, o_ref):
    o_ref[...] = jnp.maximum(
        jnp.dot(x_ref[...] + y_ref[...], w_ref[...],
                preferred_element_type=jnp.float32), 0.0)


def _add_mm_relu(x, y, w, blk):
    n = x.shape[0]
    return pl.pallas_call(
        _add_mm_relu_body,
        grid=(n // blk,),
        in_specs=[pl.BlockSpec((blk, D), lambda i: (i, 0)),
                  pl.BlockSpec((blk, D), lambda i: (i, 0)),
                  pl.BlockSpec((D, D), lambda i: (0, 0))],
        out_specs=pl.BlockSpec((blk, D), lambda i: (i, 0)),
        out_shape=jax.ShapeDtypeStruct((n, D), jnp.float32),
    )(x, y, w)


GS_NBUF = 4
GS_OCH = GS_NBUF * GS_G


def _gs_reduce_atom(buf, r0, ov, orow):
    sls = [pl.ds(j * LANES, LANES) for j in range(NSLICE)]
    accs = tuple(buf[r0, sl] + buf[r0 + 1, sl] + buf[r0 + 2, sl]
                 + buf[r0 + 3, sl] for sl in sls)

    def red(t, accs):
        b4 = r0 + t * 4
        return tuple(accs[j] + buf[b4, sls[j]] + buf[b4 + 1, sls[j]]
                     + buf[b4 + 2, sls[j]] + buf[b4 + 3, sls[j]]
                     for j in range(NSLICE))

    accs = lax.fori_loop(1, MAX_NB // 4, red, accs)
    for j in range(NSLICE):
        ov[orow, sls[j]] = accs[j]


def _gs_body(table, idxh, out, idx_v, buf0, buf1, buf2, buf3, ov0, ov1,
             sg0, sg1, sg2, sg3, so0, so1):
    c = lax.axis_index("c")
    s = lax.axis_index("s")
    w = s * NC + c
    base_atom = w * GS_A
    pltpu.sync_copy(idxh.at[pl.ds(base_atom * MAX_NB, GS_A * MAX_NB)], idx_v)

    bufs = (buf0, buf1, buf2, buf3)
    sgs = (sg0, sg1, sg2, sg3)
    ovs = (ov0, ov1)
    sos = (so0, so1)

    def issue_gather(g, b):
        pltpu.async_copy(table.at[idx_v.at[pl.ds(g * GS_R, GS_R)]],
                         bufs[b], sgs[b])

    for b in range(GS_NBUF):
        issue_gather(b, b)

    def outer(g3, _):
        for half in range(2):
            ov = ovs[half]
            orow0 = base_atom + g3 * 2 * GS_OCH + half * GS_OCH

            @pl.when(g3 > 0)
            def _():
                pltpu.make_async_copy(
                    ov, out.at[pl.ds(orow0 - 2 * GS_OCH, GS_OCH)],
                    sos[half]).wait()

            for b in range(GS_NBUF):
                g = g3 * 2 * GS_NBUF + half * GS_NBUF + b
                pltpu.make_async_copy(table.at[idx_v.at[pl.ds(0, GS_R)]],
                                      bufs[b], sgs[b]).wait()
                for a in range(GS_G):
                    _gs_reduce_atom(bufs[b], a * MAX_NB, ov, b * GS_G + a)

                @pl.when(g + GS_NBUF < GS_NG)
                def _():
                    issue_gather(g + GS_NBUF, b)

            pltpu.async_copy(ov, out.at[pl.ds(orow0, GS_OCH)], sos[half])
        return 0

    lax.fori_loop(0, GS_NG // (2 * GS_NBUF), outer, 0)
    for half in range(2):
        pltpu.make_async_copy(
            ovs[half],
            out.at[pl.ds(base_atom + GS_A - (2 - half) * GS_OCH, GS_OCH)],
            sos[half]).wait()


def _gather_sum(table, idx_flat):
    mesh = plsc.VectorSubcoreMesh(core_axis_name="c", subcore_axis_name="s")
    return pl.kernel(
        _gs_body,
        out_type=jax.ShapeDtypeStruct((N_ATOMS_PAD, D), jnp.float32),
        mesh=mesh,
        scratch_types=[
            pltpu.VMEM((GS_A * MAX_NB,), jnp.int32),
            pltpu.VMEM((GS_R, D), jnp.float32),
            pltpu.VMEM((GS_R, D), jnp.float32),
            pltpu.VMEM((GS_R, D), jnp.float32),
            pltpu.VMEM((GS_R, D), jnp.float32),
            pltpu.VMEM((GS_OCH, D), jnp.float32),
            pltpu.VMEM((GS_OCH, D), jnp.float32),
            pltpu.SemaphoreType.DMA,
            pltpu.SemaphoreType.DMA,
            pltpu.SemaphoreType.DMA,
            pltpu.SemaphoreType.DMA,
            pltpu.SemaphoreType.DMA,
            pltpu.SemaphoreType.DMA,
        ],
    )(table, idx_flat)


DG_G = 4
DG_R = DG_G * MAX_NB
DG_OCH = 8
DG_PA0 = 512
DG_PA1 = 128
DG_IDX_ROWS = NS * DG_PA0 + (NS - 1) * DG_PA1 + DG_PA0


def _dg_body(ta, tb, idxa_h, idxb_h, outa, outb,
             idxa_v, idxb_v, a0, a1, b0, b1, ova0, ova1, ovb0, ovb1,
             sa0, sa1, sb0, sb1, soa0, soa1, sob0, sob1):
    c = lax.axis_index("c")
    s = lax.axis_index("s")
    pa = jnp.where(c == 0, DG_PA0, DG_PA1)
    ng = pa // DG_G
    base_atom = jnp.where(c == 0, s * DG_PA0, NS * DG_PA0 + s * DG_PA1)
    pltpu.sync_copy(idxa_h.at[pl.ds(base_atom * MAX_NB, DG_PA0 * MAX_NB)],
                    idxa_v)
    pltpu.sync_copy(idxb_h.at[pl.ds(base_atom * MAX_NB, DG_PA0 * MAX_NB)],
                    idxb_v)

    bufa = (a0, a1)
    bufb = (b0, b1)
    sga = (sa0, sa1)
    sgb = (sb0, sb1)
    ovas = (ova0, ova1)
    ovbs = (ovb0, ovb1)
    soas = (soa0, soa1)
    sobs = (sob0, sob1)

    def issue(g, b):
        pltpu.async_copy(ta.at[idxa_v.at[pl.ds(g * DG_R, DG_R)]],
                         bufa[b], sga[b])
        pltpu.async_copy(tb.at[idxb_v.at[pl.ds(g * DG_R, DG_R)]],
                         bufb[b], sgb[b])

    issue(0, 0)
    issue(1, 1)

    def outer(g3, _):
        for half in range(2):
            ova, ovb = ovas[half], ovbs[half]
            orow0 = base_atom + g3 * 2 * DG_OCH + half * DG_OCH

            @pl.when(g3 > 0)
            def _():
                pltpu.make_async_copy(
                    ova, outa.at[pl.ds(orow0 - 2 * DG_OCH, DG_OCH)],
                    soas[half]).wait()
                pltpu.make_async_copy(
                    ovb, outb.at[pl.ds(orow0 - 2 * DG_OCH, DG_OCH)],
                    sobs[half]).wait()

            for q in range(2):
                g = g3 * 4 + half * 2 + q
                pltpu.make_async_copy(ta.at[idxa_v.at[pl.ds(0, DG_R)]],
                                      bufa[q], sga[q]).wait()
                pltpu.make_async_copy(tb.at[idxb_v.at[pl.ds(0, DG_R)]],
                                      bufb[q], sgb[q]).wait()
                for a in range(DG_G):
                    orow = q * DG_G + a
                    _gs_reduce_atom(bufa[q], a * MAX_NB, ova, orow)
                    _gs_reduce_atom(bufb[q], a * MAX_NB, ovb, orow)

                @pl.when(g + 2 < ng)
                def _():
                    issue(g + 2, q)

            pltpu.async_copy(ova, outa.at[pl.ds(orow0, DG_OCH)], soas[half])
            pltpu.async_copy(ovb, outb.at[pl.ds(orow0, DG_OCH)], sobs[half])
        return 0

    lax.fori_loop(0, ng // 4, outer, 0)
    for half in range(2):
        row = base_atom + pa - (2 - half) * DG_OCH
        pltpu.make_async_copy(ovas[half], outa.at[pl.ds(row, DG_OCH)],
                              soas[half]).wait()
        pltpu.make_async_copy(ovbs[half], outb.at[pl.ds(row, DG_OCH)],
                              sobs[half]).wait()


def _dual_gather_sum(ta, tb, idxa_flat, idxb_flat):
    mesh = plsc.VectorSubcoreMesh(core_axis_name="c", subcore_axis_name="s")
    return pl.kernel(
        _dg_body,
        out_type=(jax.ShapeDtypeStruct((N_ATOMS_PAD, D), jnp.float32),
                  jax.ShapeDtypeStruct((N_ATOMS_PAD, D), jnp.float32)),
        mesh=mesh,
        scratch_types=[
            pltpu.VMEM((DG_PA0 * MAX_NB,), jnp.int32),
            pltpu.VMEM((DG_PA0 * MAX_NB,), jnp.int32),
            pltpu.VMEM((DG_R, D), jnp.float32),
            pltpu.VMEM((DG_R, D), jnp.float32),
            pltpu.VMEM((DG_R, D), jnp.float32),
            pltpu.VMEM((DG_R, D), jnp.float32),
            pltpu.VMEM((DG_OCH, D), jnp.float32),
            pltpu.VMEM((DG_OCH, D), jnp.float32),
            pltpu.VMEM((DG_OCH, D), jnp.float32),
            pltpu.VMEM((DG_OCH, D), jnp.float32),
            pltpu.SemaphoreType.DMA,
            pltpu.SemaphoreType.DMA,
            pltpu.SemaphoreType.DMA,
            pltpu.SemaphoreType.DMA,
            pltpu.SemaphoreType.DMA,
            pltpu.SemaphoreType.DMA,
            pltpu.SemaphoreType.DMA,
            pltpu.SemaphoreType.DMA,
        ],
    )(ta, tb, idxa_flat, idxb_flat)


def _bm_body(amsg, bonds, idxa_h, idxb_h, out,
             idxa_v, idxb_v, bufa0, bufb0, bufa1, bufb1, out_v,
             sa0, sb0, sa1, sb1):
    w = lax.axis_index("s") * NC + lax.axis_index("c")
    base = w * BM_B
    pltpu.sync_copy(idxa_h.at[pl.ds(base, BM_B)], idxa_v)
    pltpu.sync_copy(idxb_h.at[pl.ds(base, BM_B)], idxb_v)
    bufsa = (bufa0, bufa1)
    bufsb = (bufb0, bufb1)
    semsa = (sa0, sa1)
    semsb = (sb0, sb1)

    def issue(g, b):
        pltpu.async_copy(amsg.at[idxa_v.at[pl.ds(g * BM_C, BM_C)]],
                         bufsa[b], semsa[b])
        pltpu.async_copy(bonds.at[idxb_v.at[pl.ds(g * BM_C, BM_C)]],
                         bufsb[b], semsb[b])

    issue(0, 0)
    issue(1, 1)

    def outer(g2, _):
        for b in range(2):
            g = g2 * 2 + b
            ba, bb = bufsa[b], bufsb[b]
            pltpu.make_async_copy(amsg.at[idxa_v.at[pl.ds(0, BM_C)]],
                                  ba, semsa[b]).wait()
            pltpu.make_async_copy(bonds.at[idxb_v.at[pl.ds(0, BM_C)]],
                                  bb, semsb[b]).wait()

            def row_body(r, _):
                for j in range(NSLICE):
                    s = pl.ds(j * LANES, LANES)
                    out_v[r, s] = jnp.maximum(ba[r, s] - bb[r, s], 0.0)
                return 0

            lax.fori_loop(0, BM_C, row_body, 0)
            pltpu.sync_copy(out_v, out.at[pl.ds(base + g * BM_C, BM_C)])

            @pl.when(g + 2 < BM_NCH)
            def _():
                issue(g + 2, b)
        return 0

    lax.fori_loop(0, BM_NCH // 2, outer, 0)


def _bond_msg(amsg, bonds, b2a, b2revb):
    mesh = plsc.VectorSubcoreMesh(core_axis_name="c", subcore_axis_name="s")
    return pl.kernel(
        _bm_body,
        out_type=jax.ShapeDtypeStruct((N_BONDS, D), jnp.float32),
        mesh=mesh,
        scratch_types=[
            pltpu.VMEM((BM_B,), jnp.int32),
            pltpu.VMEM((BM_B,), jnp.int32),
            pltpu.VMEM((BM_C, D), jnp.float32),
            pltpu.VMEM((BM_C, D), jnp.float32),
            pltpu.VMEM((BM_C, D), jnp.float32),
            pltpu.VMEM((BM_C, D), jnp.float32),
            pltpu.VMEM((BM_C, D), jnp.float32),
            pltpu.SemaphoreType.DMA,
            pltpu.SemaphoreType.DMA,
            pltpu.SemaphoreType.DMA,
            pltpu.SemaphoreType.DMA,
        ],
    )(amsg, bonds, b2a, b2revb)


def _pad_idx(idx2d):
    flat = idx2d.reshape(-1).astype(jnp.int32)
    pad = DG_IDX_ROWS * MAX_NB - flat.shape[0]
    return jnp.pad(flat, (0, pad))


def kernel(f_atoms, f_bonds, a2b, b2a, b2revb, a_scope, b_scope, a2a,
           features_batch, W_i_atom, W_h_atom, W_i_bond, W_h_bond):
    input_atom = _mm_relu(f_atoms, W_i_atom, blk=1000)
    BW = _mm2(f_bonds, W_i_bond, W_h_bond, blk=3200)
    nei_pad, AW = _dual_gather_sum(input_atom, BW,
                                   _pad_idx(a2a), _pad_idx(a2b))
    nei_a_msg = nei_pad[:N_ATOMS]
    bond_output = _bond_msg(AW, BW,
                            b2a.astype(jnp.int32), b2revb.astype(jnp.int32))

    node_output = _add_mm_relu(input_atom, nei_a_msg, W_h_atom, blk=1000)
    return (node_output, bond_output)


# speedup vs baseline: 1.1843x; 1.1037x over previous
import functools

import jax
import jax.numpy as jnp
from jax import lax
from jax.experimental import pallas as pl
from jax.experimental.pallas import tpu as pltpu
from jax.experimental.pallas import tpu_sc as plsc

D = 128
LANES = 16
NSLICE = D // LANES
NC, NS = 2, 16
NW = NC * NS

N_ATOMS = 10000
N_BONDS = 320000
MAX_NB = 32

GS_A = 320
N_ATOMS_PAD = NW * GS_A
GS_G = 1
GS_R = GS_G * MAX_NB
GS_NG = GS_A // GS_G

BM_B = N_BONDS // NW
BM_C = 40
BM_NCH = BM_B // BM_C


def _mm_relu_body(x_ref, w_ref, o_ref):
    o_ref[...] = jnp.maximum(
        jnp.dot(x_ref[...], w_ref[...], preferred_element_type=jnp.float32), 0.0)


def _mm_relu(x, w, blk):
    n = x.shape[0]
    return pl.pallas_call(
        _mm_relu_body,
        grid=(n // blk,),
        in_specs=[pl.BlockSpec((blk, D), lambda i: (i, 0)),
                  pl.BlockSpec((D, D), lambda i: (0, 0))],
        out_specs=pl.BlockSpec((blk, D), lambda i: (i, 0)),
        out_shape=jax.ShapeDtypeStruct((n, D), jnp.float32),
    )(x, w)


def _mm2_body(x_ref, wi_ref, wh_ref, o_ref):
    t = jnp.maximum(
        jnp.dot(x_ref[...], wi_ref[...], preferred_element_type=jnp.float32), 0.0)
    o_ref[...] = jnp.dot(t, wh_ref[...], preferred_element_type=jnp.float32)


def _mm2(x, wi, wh, blk):
    n = x.shape[0]
    return pl.pallas_call(
        _mm2_body,
        grid=(n // blk,),
        in_specs=[pl.BlockSpec((blk, D), lambda i: (i, 0)),
                  pl.BlockSpec((D, D), lambda i: (0, 0)),
                  pl.BlockSpec((D, D), lambda i: (0, 0))],
        out_specs=pl.BlockSpec((blk, D), lambda i: (i, 0)),
        out_shape=jax.ShapeDtypeStruct((n, D), jnp.float32),
    )(x, wi, wh)


def _add_mm_relu_body(x_ref, y_ref, w_ref, o_ref):
    o_ref[...] = jnp.maximum(
        jnp.dot(x_ref[...] + y_ref[...], w_ref[...],
                preferred_element_type=jnp.float32), 0.0)


def _add_mm_relu(x, y, w, blk):
    n = x.shape[0]
    return pl.pallas_call(
        _add_mm_relu_body,
        grid=(n // blk,),
        in_specs=[pl.BlockSpec((blk, D), lambda i: (i, 0)),
                  pl.BlockSpec((blk, D), lambda i: (i, 0)),
                  pl.BlockSpec((D, D), lambda i: (0, 0))],
        out_specs=pl.BlockSpec((blk, D), lambda i: (i, 0)),
        out_shape=jax.ShapeDtypeStruct((n, D), jnp.float32),
    )(x, y, w)


GS_NBUF = 4
GS_OCH = GS_NBUF * GS_G


def _gs_reduce_atom(buf, r0, ov, orow):
    sls = [pl.ds(j * LANES, LANES) for j in range(NSLICE)]
    accs = tuple(buf[r0, sl] + buf[r0 + 1, sl] + buf[r0 + 2, sl]
                 + buf[r0 + 3, sl] for sl in sls)

    def red(t, accs):
        b4 = r0 + t * 4
        return tuple(accs[j] + buf[b4, sls[j]] + buf[b4 + 1, sls[j]]
                     + buf[b4 + 2, sls[j]] + buf[b4 + 3, sls[j]]
                     for j in range(NSLICE))

    accs = lax.fori_loop(1, MAX_NB // 4, red, accs)
    for j in range(NSLICE):
        ov[orow, sls[j]] = accs[j]


def _gs_body(table, idxh, out, idx_v, buf0, buf1, buf2, buf3, ov0, ov1,
             sg0, sg1, sg2, sg3, so0, so1):
    c = lax.axis_index("c")
    s = lax.axis_index("s")
    w = s * NC + c
    base_atom = w * GS_A
    pltpu.sync_copy(idxh.at[pl.ds(base_atom * MAX_NB, GS_A * MAX_NB)], idx_v)

    bufs = (buf0, buf1, buf2, buf3)
    sgs = (sg0, sg1, sg2, sg3)
    ovs = (ov0, ov1)
    sos = (so0, so1)

    def issue_gather(g, b):
        pltpu.async_copy(table.at[idx_v.at[pl.ds(g * GS_R, GS_R)]],
                         bufs[b], sgs[b])

    for b in range(GS_NBUF):
        issue_gather(b, b)

    def outer(g3, _):
        for half in range(2):
            ov = ovs[half]
            orow0 = base_atom + g3 * 2 * GS_OCH + half * GS_OCH

            @pl.when(g3 > 0)
            def _():
                pltpu.make_async_copy(
                    ov, out.at[pl.ds(orow0 - 2 * GS_OCH, GS_OCH)],
                    sos[half]).wait()

            for b in range(GS_NBUF):
                g = g3 * 2 * GS_NBUF + half * GS_NBUF + b
                pltpu.make_async_copy(table.at[idx_v.at[pl.ds(0, GS_R)]],
                                      bufs[b], sgs[b]).wait()
                for a in range(GS_G):
                    _gs_reduce_atom(bufs[b], a * MAX_NB, ov, b * GS_G + a)

                @pl.when(g + GS_NBUF < GS_NG)
                def _():
                    issue_gather(g + GS_NBUF, b)

            pltpu.async_copy(ov, out.at[pl.ds(orow0, GS_OCH)], sos[half])
        return 0

    lax.fori_loop(0, GS_NG // (2 * GS_NBUF), outer, 0)
    for half in range(2):
        pltpu.make_async_copy(
            ovs[half],
            out.at[pl.ds(base_atom + GS_A - (2 - half) * GS_OCH, GS_OCH)],
            sos[half]).wait()


def _gather_sum(table, idx_flat):
    mesh = plsc.VectorSubcoreMesh(core_axis_name="c", subcore_axis_name="s")
    return pl.kernel(
        _gs_body,
        out_type=jax.ShapeDtypeStruct((N_ATOMS_PAD, D), jnp.float32),
        mesh=mesh,
        scratch_types=[
            pltpu.VMEM((GS_A * MAX_NB,), jnp.int32),
            pltpu.VMEM((GS_R, D), jnp.float32),
            pltpu.VMEM((GS_R, D), jnp.float32),
            pltpu.VMEM((GS_R, D), jnp.float32),
            pltpu.VMEM((GS_R, D), jnp.float32),
            pltpu.VMEM((GS_OCH, D), jnp.float32),
            pltpu.VMEM((GS_OCH, D), jnp.float32),
            pltpu.SemaphoreType.DMA,
            pltpu.SemaphoreType.DMA,
            pltpu.SemaphoreType.DMA,
            pltpu.SemaphoreType.DMA,
            pltpu.SemaphoreType.DMA,
            pltpu.SemaphoreType.DMA,
        ],
    )(table, idx_flat)


DG_G = 4
DG_R = DG_G * MAX_NB
DG_OCH = 8
DG_PA0 = 576
DG_PA1 = 64
DG_IDX_ROWS = NS * DG_PA0 + (NS - 1) * DG_PA1 + DG_PA0


def _dg_body(ta, tb, idxa_h, idxb_h, outa, outb,
             idxa_v, idxb_v, a0, a1, b0, b1, ova0, ova1, ovb0, ovb1,
             sa0, sa1, sb0, sb1, soa0, soa1, sob0, sob1):
    c = lax.axis_index("c")
    s = lax.axis_index("s")
    pa = jnp.where(c == 0, DG_PA0, DG_PA1)
    ng = pa // DG_G
    base_atom = jnp.where(c == 0, s * DG_PA0, NS * DG_PA0 + s * DG_PA1)
    pltpu.sync_copy(idxa_h.at[pl.ds(base_atom * MAX_NB, DG_PA0 * MAX_NB)],
                    idxa_v)
    pltpu.sync_copy(idxb_h.at[pl.ds(base_atom * MAX_NB, DG_PA0 * MAX_NB)],
                    idxb_v)

    bufa = (a0, a1)
    bufb = (b0, b1)
    sga = (sa0, sa1)
    sgb = (sb0, sb1)
    ovas = (ova0, ova1)
    ovbs = (ovb0, ovb1)
    soas = (soa0, soa1)
    sobs = (sob0, sob1)

    def issue(g, b):
        pltpu.async_copy(ta.at[idxa_v.at[pl.ds(g * DG_R, DG_R)]],
                         bufa[b], sga[b])
        pltpu.async_copy(tb.at[idxb_v.at[pl.ds(g * DG_R, DG_R)]],
                         bufb[b], sgb[b])

    issue(0, 0)
    issue(1, 1)

    def outer(g3, _):
        for half in range(2):
            ova, ovb = ovas[half], ovbs[half]
            orow0 = base_atom + g3 * 2 * DG_OCH + half * DG_OCH

            @pl.when(g3 > 0)
            def _():
                pltpu.make_async_copy(
                    ova, outa.at[pl.ds(orow0 - 2 * DG_OCH, DG_OCH)],
                    soas[half]).wait()
                pltpu.make_async_copy(
                    ovb, outb.at[pl.ds(orow0 - 2 * DG_OCH, DG_OCH)],
                    sobs[half]).wait()

            for q in range(2):
                g = g3 * 4 + half * 2 + q
                pltpu.make_async_copy(ta.at[idxa_v.at[pl.ds(0, DG_R)]],
                                      bufa[q], sga[q]).wait()
                pltpu.make_async_copy(tb.at[idxb_v.at[pl.ds(0, DG_R)]],
                                      bufb[q], sgb[q]).wait()
                for a in range(DG_G):
                    orow = q * DG_G + a
                    _gs_reduce_atom(bufa[q], a * MAX_NB, ova, orow)
                    _gs_reduce_atom(bufb[q], a * MAX_NB, ovb, orow)

                @pl.when(g + 2 < ng)
                def _():
                    issue(g + 2, q)

            pltpu.async_copy(ova, outa.at[pl.ds(orow0, DG_OCH)], soas[half])
            pltpu.async_copy(ovb, outb.at[pl.ds(orow0, DG_OCH)], sobs[half])
        return 0

    lax.fori_loop(0, ng // 4, outer, 0)
    for half in range(2):
        row = base_atom + pa - (2 - half) * DG_OCH
        pltpu.make_async_copy(ovas[half], outa.at[pl.ds(row, DG_OCH)],
                              soas[half]).wait()
        pltpu.make_async_copy(ovbs[half], outb.at[pl.ds(row, DG_OCH)],
                              sobs[half]).wait()


def _dual_gather_sum(ta, tb, idxa_flat, idxb_flat):
    mesh = plsc.VectorSubcoreMesh(core_axis_name="c", subcore_axis_name="s")
    return pl.kernel(
        _dg_body,
        out_type=(jax.ShapeDtypeStruct((N_ATOMS_PAD, D), jnp.float32),
                  jax.ShapeDtypeStruct((N_ATOMS_PAD, D), jnp.float32)),
        mesh=mesh,
        scratch_types=[
            pltpu.VMEM((DG_PA0 * MAX_NB,), jnp.int32),
            pltpu.VMEM((DG_PA0 * MAX_NB,), jnp.int32),
            pltpu.VMEM((DG_R, D), jnp.float32),
            pltpu.VMEM((DG_R, D), jnp.float32),
            pltpu.VMEM((DG_R, D), jnp.float32),
            pltpu.VMEM((DG_R, D), jnp.float32),
            pltpu.VMEM((DG_OCH, D), jnp.float32),
            pltpu.VMEM((DG_OCH, D), jnp.float32),
            pltpu.VMEM((DG_OCH, D), jnp.float32),
            pltpu.VMEM((DG_OCH, D), jnp.float32),
            pltpu.SemaphoreType.DMA,
            pltpu.SemaphoreType.DMA,
            pltpu.SemaphoreType.DMA,
            pltpu.SemaphoreType.DMA,
            pltpu.SemaphoreType.DMA,
            pltpu.SemaphoreType.DMA,
            pltpu.SemaphoreType.DMA,
            pltpu.SemaphoreType.DMA,
        ],
    )(ta, tb, idxa_flat, idxb_flat)


def _bm_body(amsg, bonds, idxa_h, idxb_h, out,
             idxa_v, idxb_v, bufa0, bufb0, bufa1, bufb1, out_v,
             sa0, sb0, sa1, sb1):
    w = lax.axis_index("s") * NC + lax.axis_index("c")
    base = w * BM_B
    pltpu.sync_copy(idxa_h.at[pl.ds(base, BM_B)], idxa_v)
    pltpu.sync_copy(idxb_h.at[pl.ds(base, BM_B)], idxb_v)
    bufsa = (bufa0, bufa1)
    bufsb = (bufb0, bufb1)
    semsa = (sa0, sa1)
    semsb = (sb0, sb1)

    def issue(g, b):
        pltpu.async_copy(amsg.at[idxa_v.at[pl.ds(g * BM_C, BM_C)]],
                         bufsa[b], semsa[b])
        pltpu.async_copy(bonds.at[idxb_v.at[pl.ds(g * BM_C, BM_C)]],
                         bufsb[b], semsb[b])

    issue(0, 0)
    issue(1, 1)

    def outer(g2, _):
        for b in range(2):
            g = g2 * 2 + b
            ba, bb = bufsa[b], bufsb[b]
            pltpu.make_async_copy(amsg.at[idxa_v.at[pl.ds(0, BM_C)]],
                                  ba, semsa[b]).wait()
            pltpu.make_async_copy(bonds.at[idxb_v.at[pl.ds(0, BM_C)]],
                                  bb, semsb[b]).wait()

            def row_body(r, _):
                for j in range(NSLICE):
                    s = pl.ds(j * LANES, LANES)
                    out_v[r, s] = jnp.maximum(ba[r, s] - bb[r, s], 0.0)
                return 0

            lax.fori_loop(0, BM_C, row_body, 0)
            pltpu.sync_copy(out_v, out.at[pl.ds(base + g * BM_C, BM_C)])

            @pl.when(g + 2 < BM_NCH)
            def _():
                issue(g + 2, b)
        return 0

    lax.fori_loop(0, BM_NCH // 2, outer, 0)


def _bond_msg(amsg, bonds, b2a, b2revb):
    mesh = plsc.VectorSubcoreMesh(core_axis_name="c", subcore_axis_name="s")
    return pl.kernel(
        _bm_body,
        out_type=jax.ShapeDtypeStruct((N_BONDS, D), jnp.float32),
        mesh=mesh,
        scratch_types=[
            pltpu.VMEM((BM_B,), jnp.int32),
            pltpu.VMEM((BM_B,), jnp.int32),
            pltpu.VMEM((BM_C, D), jnp.float32),
            pltpu.VMEM((BM_C, D), jnp.float32),
            pltpu.VMEM((BM_C, D), jnp.float32),
            pltpu.VMEM((BM_C, D), jnp.float32),
            pltpu.VMEM((BM_C, D), jnp.float32),
            pltpu.SemaphoreType.DMA,
            pltpu.SemaphoreType.DMA,
            pltpu.SemaphoreType.DMA,
            pltpu.SemaphoreType.DMA,
        ],
    )(amsg, bonds, b2a, b2revb)


def _pad_idx(idx2d):
    flat = idx2d.reshape(-1).astype(jnp.int32)
    pad = DG_IDX_ROWS * MAX_NB - flat.shape[0]
    return jnp.pad(flat, (0, pad))


def kernel(f_atoms, f_bonds, a2b, b2a, b2revb, a_scope, b_scope, a2a,
           features_batch, W_i_atom, W_h_atom, W_i_bond, W_h_bond):
    input_atom = _mm_relu(f_atoms, W_i_atom, blk=1000)
    BW = _mm2(f_bonds, W_i_bond, W_h_bond, blk=3200)
    nei_pad, AW = _dual_gather_sum(input_atom, BW,
                                   _pad_idx(a2a), _pad_idx(a2b))
    nei_a_msg = nei_pad[:N_ATOMS]
    bond_output = _bond_msg(AW, BW,
                            b2a.astype(jnp.int32), b2revb.astype(jnp.int32))

    node_output = _add_mm_relu(input_atom, nei_a_msg, W_h_atom, blk=1000)
    return (node_output, bond_output)


# speedup vs baseline: 1.1902x; 1.0050x over previous
import functools

import jax
import jax.numpy as jnp
from jax import lax
from jax.experimental import pallas as pl
from jax.experimental.pallas import tpu as pltpu
from jax.experimental.pallas import tpu_sc as plsc

D = 128
LANES = 16
NSLICE = D // LANES
NC, NS = 2, 16
NW = NC * NS

N_ATOMS = 10000
N_BONDS = 320000
MAX_NB = 32

GS_A = 320
N_ATOMS_PAD = NW * GS_A
GS_G = 1
GS_R = GS_G * MAX_NB
GS_NG = GS_A // GS_G

BM_B = N_BONDS // NW
BM_C = 40
BM_NCH = BM_B // BM_C


def _mm_relu_body(x_ref, w_ref, o_ref):
    o_ref[...] = jnp.maximum(
        jnp.dot(x_ref[...], w_ref[...], preferred_element_type=jnp.float32), 0.0)


def _mm_relu(x, w, blk):
    n = x.shape[0]
    return pl.pallas_call(
        _mm_relu_body,
        grid=(n // blk,),
        in_specs=[pl.BlockSpec((blk, D), lambda i: (i, 0)),
                  pl.BlockSpec((D, D), lambda i: (0, 0))],
        out_specs=pl.BlockSpec((blk, D), lambda i: (i, 0)),
        out_shape=jax.ShapeDtypeStruct((n, D), jnp.float32),
    )(x, w)


def _mm2_body(x_ref, wi_ref, wh_ref, o_ref):
    t = jnp.maximum(
        jnp.dot(x_ref[...], wi_ref[...], preferred_element_type=jnp.float32), 0.0)
    o_ref[...] = jnp.dot(t, wh_ref[...], preferred_element_type=jnp.float32)


def _mm2(x, wi, wh, blk):
    n = x.shape[0]
    return pl.pallas_call(
        _mm2_body,
        grid=(n // blk,),
        in_specs=[pl.BlockSpec((blk, D), lambda i: (i, 0)),
                  pl.BlockSpec((D, D), lambda i: (0, 0)),
                  pl.BlockSpec((D, D), lambda i: (0, 0))],
        out_specs=pl.BlockSpec((blk, D), lambda i: (i, 0)),
        out_shape=jax.ShapeDtypeStruct((n, D), jnp.float32),
    )(x, wi, wh)


def _add_mm_relu_body(x_ref, y_ref, w_ref, o_ref):
    o_ref[...] = jnp.maximum(
        jnp.dot(x_ref[...] + y_ref[...], w_ref[...],
                preferred_element_type=jnp.float32), 0.0)


def _add_mm_relu(x, y, w, blk):
    n = x.shape[0]
    return pl.pallas_call(
        _add_mm_relu_body,
        grid=(n // blk,),
        in_specs=[pl.BlockSpec((blk, D), lambda i: (i, 0)),
                  pl.BlockSpec((blk, D), lambda i: (i, 0)),
                  pl.BlockSpec((D, D), lambda i: (0, 0))],
        out_specs=pl.BlockSpec((blk, D), lambda i: (i, 0)),
        out_shape=jax.ShapeDtypeStruct((n, D), jnp.float32),
    )(x, y, w)


GS_NBUF = 4
GS_OCH = GS_NBUF * GS_G


def _gs_reduce_atom(buf, r0, ov, orow):
    sls = [pl.ds(j * LANES, LANES) for j in range(NSLICE)]
    accs = tuple(buf[r0, sl] + buf[r0 + 1, sl] + buf[r0 + 2, sl]
                 + buf[r0 + 3, sl] for sl in sls)

    def red(t, accs):
        b4 = r0 + t * 4
        return tuple(accs[j] + buf[b4, sls[j]] + buf[b4 + 1, sls[j]]
                     + buf[b4 + 2, sls[j]] + buf[b4 + 3, sls[j]]
                     for j in range(NSLICE))

    accs = lax.fori_loop(1, MAX_NB // 4, red, accs)
    for j in range(NSLICE):
        ov[orow, sls[j]] = accs[j]


def _gs_body(table, idxh, out, idx_v, buf0, buf1, buf2, buf3, ov0, ov1,
             sg0, sg1, sg2, sg3, so0, so1):
    c = lax.axis_index("c")
    s = lax.axis_index("s")
    w = s * NC + c
    base_atom = w * GS_A
    pltpu.sync_copy(idxh.at[pl.ds(base_atom * MAX_NB, GS_A * MAX_NB)], idx_v)

    bufs = (buf0, buf1, buf2, buf3)
    sgs = (sg0, sg1, sg2, sg3)
    ovs = (ov0, ov1)
    sos = (so0, so1)

    def issue_gather(g, b):
        pltpu.async_copy(table.at[idx_v.at[pl.ds(g * GS_R, GS_R)]],
                         bufs[b], sgs[b])

    for b in range(GS_NBUF):
        issue_gather(b, b)

    def outer(g3, _):
        for half in range(2):
            ov = ovs[half]
            orow0 = base_atom + g3 * 2 * GS_OCH + half * GS_OCH

            @pl.when(g3 > 0)
            def _():
                pltpu.make_async_copy(
                    ov, out.at[pl.ds(orow0 - 2 * GS_OCH, GS_OCH)],
                    sos[half]).wait()

            for b in range(GS_NBUF):
                g = g3 * 2 * GS_NBUF + half * GS_NBUF + b
                pltpu.make_async_copy(table.at[idx_v.at[pl.ds(0, GS_R)]],
                                      bufs[b], sgs[b]).wait()
                for a in range(GS_G):
                    _gs_reduce_atom(bufs[b], a * MAX_NB, ov, b * GS_G + a)

                @pl.when(g + GS_NBUF < GS_NG)
                def _():
                    issue_gather(g + GS_NBUF, b)

            pltpu.async_copy(ov, out.at[pl.ds(orow0, GS_OCH)], sos[half])
        return 0

    lax.fori_loop(0, GS_NG // (2 * GS_NBUF), outer, 0)
    for half in range(2):
        pltpu.make_async_copy(
            ovs[half],
            out.at[pl.ds(base_atom + GS_A - (2 - half) * GS_OCH, GS_OCH)],
            sos[half]).wait()


def _gather_sum(table, idx_flat):
    mesh = plsc.VectorSubcoreMesh(core_axis_name="c", subcore_axis_name="s")
    return pl.kernel(
        _gs_body,
        out_type=jax.ShapeDtypeStruct((N_ATOMS_PAD, D), jnp.float32),
        mesh=mesh,
        scratch_types=[
            pltpu.VMEM((GS_A * MAX_NB,), jnp.int32),
            pltpu.VMEM((GS_R, D), jnp.float32),
            pltpu.VMEM((GS_R, D), jnp.float32),
            pltpu.VMEM((GS_R, D), jnp.float32),
            pltpu.VMEM((GS_R, D), jnp.float32),
            pltpu.VMEM((GS_OCH, D), jnp.float32),
            pltpu.VMEM((GS_OCH, D), jnp.float32),
            pltpu.SemaphoreType.DMA,
            pltpu.SemaphoreType.DMA,
            pltpu.SemaphoreType.DMA,
            pltpu.SemaphoreType.DMA,
            pltpu.SemaphoreType.DMA,
            pltpu.SemaphoreType.DMA,
        ],
    )(table, idx_flat)


DG_G = 4
DG_R = DG_G * MAX_NB
DG_OCH = 8
DG_PA0 = 608
DG_PA1 = 32
DG_IDX_ROWS = NS * DG_PA0 + (NS - 1) * DG_PA1 + DG_PA0


def _dg_body(ta, tb, idxa_h, idxb_h, outa, outb,
             idxa_v, idxb_v, a0, a1, b0, b1, ova0, ova1, ovb0, ovb1,
             sa0, sa1, sb0, sb1, soa0, soa1, sob0, sob1):
    c = lax.axis_index("c")
    s = lax.axis_index("s")
    pa = jnp.where(c == 0, DG_PA0, DG_PA1)
    ng = pa // DG_G
    base_atom = jnp.where(c == 0, s * DG_PA0, NS * DG_PA0 + s * DG_PA1)
    pltpu.sync_copy(idxa_h.at[pl.ds(base_atom * MAX_NB, DG_PA0 * MAX_NB)],
                    idxa_v)
    pltpu.sync_copy(idxb_h.at[pl.ds(base_atom * MAX_NB, DG_PA0 * MAX_NB)],
                    idxb_v)

    bufa = (a0, a1)
    bufb = (b0, b1)
    sga = (sa0, sa1)
    sgb = (sb0, sb1)
    ovas = (ova0, ova1)
    ovbs = (ovb0, ovb1)
    soas = (soa0, soa1)
    sobs = (sob0, sob1)

    def issue(g, b):
        pltpu.async_copy(ta.at[idxa_v.at[pl.ds(g * DG_R, DG_R)]],
                         bufa[b], sga[b])
        pltpu.async_copy(tb.at[idxb_v.at[pl.ds(g * DG_R, DG_R)]],
                         bufb[b], sgb[b])

    issue(0, 0)
    issue(1, 1)

    def outer(g3, _):
        for half in range(2):
            ova, ovb = ovas[half], ovbs[half]
            orow0 = base_atom + g3 * 2 * DG_OCH + half * DG_OCH

            @pl.when(g3 > 0)
            def _():
                pltpu.make_async_copy(
                    ova, outa.at[pl.ds(orow0 - 2 * DG_OCH, DG_OCH)],
                    soas[half]).wait()
                pltpu.make_async_copy(
                    ovb, outb.at[pl.ds(orow0 - 2 * DG_OCH, DG_OCH)],
                    sobs[half]).wait()

            for q in range(2):
                g = g3 * 4 + half * 2 + q
                pltpu.make_async_copy(ta.at[idxa_v.at[pl.ds(0, DG_R)]],
                                      bufa[q], sga[q]).wait()
                pltpu.make_async_copy(tb.at[idxb_v.at[pl.ds(0, DG_R)]],
                                      bufb[q], sgb[q]).wait()
                for a in range(DG_G):
                    orow = q * DG_G + a
                    _gs_reduce_atom(bufa[q], a * MAX_NB, ova, orow)
                    _gs_reduce_atom(bufb[q], a * MAX_NB, ovb, orow)

                @pl.when(g + 2 < ng)
                def _():
                    issue(g + 2, q)

            pltpu.async_copy(ova, outa.at[pl.ds(orow0, DG_OCH)], soas[half])
            pltpu.async_copy(ovb, outb.at[pl.ds(orow0, DG_OCH)], sobs[half])
        return 0

    lax.fori_loop(0, ng // 4, outer, 0)
    for half in range(2):
        row = base_atom + pa - (2 - half) * DG_OCH
        pltpu.make_async_copy(ovas[half], outa.at[pl.ds(row, DG_OCH)],
                              soas[half]).wait()
        pltpu.make_async_copy(ovbs[half], outb.at[pl.ds(row, DG_OCH)],
                              sobs[half]).wait()


def _dual_gather_sum(ta, tb, idxa_flat, idxb_flat):
    mesh = plsc.VectorSubcoreMesh(core_axis_name="c", subcore_axis_name="s")
    return pl.kernel(
        _dg_body,
        out_type=(jax.ShapeDtypeStruct((N_ATOMS_PAD, D), jnp.float32),
                  jax.ShapeDtypeStruct((N_ATOMS_PAD, D), jnp.float32)),
        mesh=mesh,
        scratch_types=[
            pltpu.VMEM((DG_PA0 * MAX_NB,), jnp.int32),
            pltpu.VMEM((DG_PA0 * MAX_NB,), jnp.int32),
            pltpu.VMEM((DG_R, D), jnp.float32),
            pltpu.VMEM((DG_R, D), jnp.float32),
            pltpu.VMEM((DG_R, D), jnp.float32),
            pltpu.VMEM((DG_R, D), jnp.float32),
            pltpu.VMEM((DG_OCH, D), jnp.float32),
            pltpu.VMEM((DG_OCH, D), jnp.float32),
            pltpu.VMEM((DG_OCH, D), jnp.float32),
            pltpu.VMEM((DG_OCH, D), jnp.float32),
            pltpu.SemaphoreType.DMA,
            pltpu.SemaphoreType.DMA,
            pltpu.SemaphoreType.DMA,
            pltpu.SemaphoreType.DMA,
            pltpu.SemaphoreType.DMA,
            pltpu.SemaphoreType.DMA,
            pltpu.SemaphoreType.DMA,
            pltpu.SemaphoreType.DMA,
        ],
    )(ta, tb, idxa_flat, idxb_flat)


def _bm_body(amsg, bonds, idxa_h, idxb_h, out,
             idxa_v, idxb_v, bufa0, bufb0, bufa1, bufb1, out_v,
             sa0, sb0, sa1, sb1):
    w = lax.axis_index("s") * NC + lax.axis_index("c")
    base = w * BM_B
    pltpu.sync_copy(idxa_h.at[pl.ds(base, BM_B)], idxa_v)
    pltpu.sync_copy(idxb_h.at[pl.ds(base, BM_B)], idxb_v)
    bufsa = (bufa0, bufa1)
    bufsb = (bufb0, bufb1)
    semsa = (sa0, sa1)
    semsb = (sb0, sb1)

    def issue(g, b):
        pltpu.async_copy(amsg.at[idxa_v.at[pl.ds(g * BM_C, BM_C)]],
                         bufsa[b], semsa[b])
        pltpu.async_copy(bonds.at[idxb_v.at[pl.ds(g * BM_C, BM_C)]],
                         bufsb[b], semsb[b])

    issue(0, 0)
    issue(1, 1)

    def outer(g2, _):
        for b in range(2):
            g = g2 * 2 + b
            ba, bb = bufsa[b], bufsb[b]
            pltpu.make_async_copy(amsg.at[idxa_v.at[pl.ds(0, BM_C)]],
                                  ba, semsa[b]).wait()
            pltpu.make_async_copy(bonds.at[idxb_v.at[pl.ds(0, BM_C)]],
                                  bb, semsb[b]).wait()

            def row_body(r, _):
                for j in range(NSLICE):
                    s = pl.ds(j * LANES, LANES)
                    out_v[r, s] = jnp.maximum(ba[r, s] - bb[r, s], 0.0)
                return 0

            lax.fori_loop(0, BM_C, row_body, 0)
            pltpu.sync_copy(out_v, out.at[pl.ds(base + g * BM_C, BM_C)])

            @pl.when(g + 2 < BM_NCH)
            def _():
                issue(g + 2, b)
        return 0

    lax.fori_loop(0, BM_NCH // 2, outer, 0)


def _bond_msg(amsg, bonds, b2a, b2revb):
    mesh = plsc.VectorSubcoreMesh(core_axis_name="c", subcore_axis_name="s")
    return pl.kernel(
        _bm_body,
        out_type=jax.ShapeDtypeStruct((N_BONDS, D), jnp.float32),
        mesh=mesh,
        scratch_types=[
            pltpu.VMEM((BM_B,), jnp.int32),
            pltpu.VMEM((BM_B,), jnp.int32),
            pltpu.VMEM((BM_C, D), jnp.float32),
            pltpu.VMEM((BM_C, D), jnp.float32),
            pltpu.VMEM((BM_C, D), jnp.float32),
            pltpu.VMEM((BM_C, D), jnp.float32),
            pltpu.VMEM((BM_C, D), jnp.float32),
            pltpu.SemaphoreType.DMA,
            pltpu.SemaphoreType.DMA,
            pltpu.SemaphoreType.DMA,
            pltpu.SemaphoreType.DMA,
        ],
    )(amsg, bonds, b2a, b2revb)


def _pad_idx(idx2d):
    flat = idx2d.reshape(-1).astype(jnp.int32)
    pad = DG_IDX_ROWS * MAX_NB - flat.shape[0]
    return jnp.pad(flat, (0, pad))


def kernel(f_atoms, f_bonds, a2b, b2a, b2revb, a_scope, b_scope, a2a,
           features_batch, W_i_atom, W_h_atom, W_i_bond, W_h_bond):
    input_atom = _mm_relu(f_atoms, W_i_atom, blk=1000)
    BW = _mm2(f_bonds, W_i_bond, W_h_bond, blk=3200)
    nei_pad, AW = _dual_gather_sum(input_atom, BW,
                                   _pad_idx(a2a), _pad_idx(a2b))
    nei_a_msg = nei_pad[:N_ATOMS]
    bond_output = _bond_msg(AW, BW,
                            b2a.astype(jnp.int32), b2revb.astype(jnp.int32))

    node_output = _add_mm_relu(input_atom, nei_a_msg, W_h_atom, blk=1000)
    return (node_output, bond_output)
